# baseline (speedup 1.0000x reference)
import functools
import math

import jax
import jax.numpy as jnp
from jax.experimental import pallas as pl
from jax.experimental.pallas import tpu as pltpu

N = 10000
NP = 10240
K1, K1P = 5000, 5120
K2, K2P = 2500, 2560
K3, K3P = 1250, 1280
F = 128
RP1 = (512, 512, 1024)
RP2 = (512, 512, 1024)
RP3 = (256, 256, 512)
ZBM = 512
ABM, ABK = 512, 256


def _rp_body(a_ref, b_ref, m_ref, mt_ref, deg_ref, acc_ref, *, nk, bm, bn):
    j = pl.program_id(0)
    i = pl.program_id(1)
    k = pl.program_id(2)

    @pl.when(k == 0)
    def _zero():
        acc_ref[...] = jnp.zeros_like(acc_ref)

    acc_ref[...] += jax.lax.dot_general(
        a_ref[...], b_ref[...], (((1,), (1,)), ((), ())),
        preferred_element_type=jnp.float32)

    @pl.when(k == nk - 1)
    def _fin():
        acc = acc_ref[...]
        rows = i * bm + jax.lax.broadcasted_iota(jnp.int32, (bm, bn), 0)
        cols = j * bn + jax.lax.broadcasted_iota(jnp.int32, (bm, bn), 1)
        acc = jnp.where(rows == cols, 0.0, acc)
        m_ref[...] = acc.astype(m_ref.dtype)
        mt_ref[...] = acc.T.astype(mt_ref.dtype)

        @pl.when(i == 0)
        def _zd():
            deg_ref[...] = jnp.zeros_like(deg_ref)

        deg_ref[...] += jnp.sum(acc, axis=0, keepdims=True)


def _restricted_product(a, bt, bm, bn, bk):
    M, K = a.shape
    Nc = bt.shape[0]
    assert M % bm == 0 and Nc % bn == 0 and K % bk == 0, (M, Nc, K, bm, bn, bk)
    nk = K // bk
    grid = (Nc // bn, M // bm, nk)
    return pl.pallas_call(
        functools.partial(_rp_body, nk=nk, bm=bm, bn=bn),
        grid=grid,
        in_specs=[
            pl.BlockSpec((bm, bk), lambda j, i, k: (i, k)),
            pl.BlockSpec((bn, bk), lambda j, i, k: (j, k)),
        ],
        out_specs=[
            pl.BlockSpec((bm, bn), lambda j, i, k: (i, j)),
            pl.BlockSpec((bn, bm), lambda j, i, k: (j, i)),
            pl.BlockSpec((1, bn), lambda j, i, k: (0, j)),
        ],
        out_shape=[
            jax.ShapeDtypeStruct((M, Nc), jnp.bfloat16),
            jax.ShapeDtypeStruct((Nc, M), jnp.bfloat16),
            jax.ShapeDtypeStruct((1, Nc), jnp.float32),
        ],
        scratch_shapes=[pltpu.VMEM((bm, bn), jnp.float32)],
    )(a, bt)


def _z_body(feat_ref, w_ref, b_ref, dis_ref, wv_ref, zs_ref, e_ref):
    z = jnp.dot(feat_ref[...], w_ref[...], preferred_element_type=jnp.float32)
    dis = dis_ref[...]
    zs = dis * z
    zs_ref[...] = zs
    e_ref[...] = wv_ref[...] * dis * zs + b_ref[...]


def _z_producer(feat, w, b, dis, wvec):
    M, Fin = feat.shape
    bm = ZBM if M % ZBM == 0 else 256
    assert M % bm == 0, (M, bm)
    return pl.pallas_call(
        _z_body,
        grid=(M // bm,),
        in_specs=[
            pl.BlockSpec((bm, Fin), lambda i: (i, 0)),
            pl.BlockSpec((Fin, F), lambda i: (0, 0)),
            pl.BlockSpec((1, F), lambda i: (0, 0)),
            pl.BlockSpec((bm, 1), lambda i: (i, 0)),
            pl.BlockSpec((bm, 1), lambda i: (i, 0)),
        ],
        out_specs=[
            pl.BlockSpec((bm, F), lambda i: (i, 0)),
            pl.BlockSpec((bm, F), lambda i: (i, 0)),
        ],
        out_shape=[
            jax.ShapeDtypeStruct((M, F), jnp.float32),
            jax.ShapeDtypeStruct((M, F), jnp.float32),
        ],
    )(feat, w, b.reshape(1, F), dis.reshape(M, 1), wvec.reshape(M, 1))


def _agg_body(mt_ref, zs_ref, dis_ref, e_ref, out_ref, acc_ref, *, nk, relu):
    k = pl.program_id(1)

    @pl.when(k == 0)
    def _zero():
        acc_ref[...] = jnp.zeros_like(acc_ref)

    acc_ref[...] += jnp.dot(mt_ref[...].astype(jnp.float32), zs_ref[...],
                            preferred_element_type=jnp.float32)

    @pl.when(k == nk - 1)
    def _fin():
        o = dis_ref[...] * acc_ref[...] + e_ref[...]
        if relu:
            o = jnp.maximum(o, 0.0)
        out_ref[...] = o


def _aggregate(mt, zs, dis, e, relu):
    M, K = mt.shape
    bm = ABM if M % ABM == 0 else 256
    bk = ABK
    assert M % bm == 0 and K % bk == 0, (M, K, bm, bk)
    nk = K // bk
    return pl.pallas_call(
        functools.partial(_agg_body, nk=nk, relu=relu),
        grid=(M // bm, nk),
        in_specs=[
            pl.BlockSpec((bm, bk), lambda i, k: (i, k)),
            pl.BlockSpec((bk, F), lambda i, k: (k, 0)),
            pl.BlockSpec((bm, 1), lambda i, k: (i, 0)),
            pl.BlockSpec((bm, F), lambda i, k: (i, 0)),
        ],
        out_specs=pl.BlockSpec((bm, F), lambda i, k: (i, 0)),
        out_shape=jax.ShapeDtypeStruct((M, F), jnp.float32),
        scratch_shapes=[pltpu.VMEM((bm, F), jnp.float32)],
    )(mt, zs, dis.reshape(M, 1), e)


def _topk(score, k, kp, dummy_idx):
    thr = jax.lax.top_k(score, k)[0][-1]
    mask = score >= thr
    perm = jnp.nonzero(mask, size=kp, fill_value=dummy_idx)[0].astype(jnp.int32)
    valid = jnp.arange(kp) < k
    vals = jnp.where(valid, score[perm], 0.0)
    return perm, vals


def _gather_diag(s_mat, perm, k, kp):
    g = s_mat[perm]
    ind = jnp.where(jnp.arange(kp) < k, 1.0, 0.0).astype(s_mat.dtype)
    return g.at[jnp.arange(kp), perm].add(ind)


def kernel(x, edge_index, Wd0, bd0, Wd1, bd1, Wd2, bd2, Wd3, bd3,
           attn0, attn1, attn2, Wu0, bu0, Wu1, bu1, Wu2, bu2):
    src, dst = edge_index[0], edge_index[1]

    xpad = jnp.zeros((NP, F), jnp.float32).at[:N].set(x)

    ones_e = jnp.ones_like(src, dtype=jnp.float32)
    indeg = jnp.zeros((NP,), jnp.float32).at[dst].add(ones_e)
    selfcnt = jnp.zeros((NP,), jnp.float32).at[dst].add(
        jnp.where(src == dst, 1.0, 0.0))
    deg0 = indeg + jnp.where(selfcnt > 0, 0.0, 2.0)
    dinv0 = jax.lax.rsqrt(deg0)
    wvec0 = jnp.where(selfcnt > 0, 0.0, 2.0)

    zs0, e0 = _z_producer(xpad, Wd0, bd0, dinv0, wvec0)
    t0 = jnp.zeros((NP, F), jnp.float32).at[dst].add(zs0[src])
    x1 = jax.nn.relu(dinv0[:, None] * t0 + e0)

    s1 = jnp.tanh((x1 @ attn0) / jnp.linalg.norm(attn0))
    s1 = jnp.where(jnp.arange(NP) < N, s1, -2.0)
    p1, v1 = _topk(s1, K1, K1P, NP - 1)
    xp1 = x1[p1] * v1[:, None]

    inv1 = jnp.full((NP,), -1, jnp.int32).at[p1].set(
        jnp.arange(K1P, dtype=jnp.int32))
    offd = src != dst
    bf = jnp.bfloat16
    msrc = offd & (inv1[src] >= 0) & (inv1[src] < K1)
    Rm = jnp.zeros((K1P, NP), bf).at[
        jnp.where(msrc, inv1[src], 0), dst].add(jnp.where(msrc, 1.0, 0.0).astype(bf))
    Rm = Rm.at[jnp.arange(K1P), p1].add(
        jnp.where(jnp.arange(K1P) < K1, 1.0, 0.0).astype(bf))
    mdst = offd & (inv1[dst] >= 0) & (inv1[dst] < K1)
    Ct = jnp.zeros((K1P, NP), bf).at[
        jnp.where(mdst, inv1[dst], 0), src].add(jnp.where(mdst, 1.0, 0.0).astype(bf))
    Ct = Ct.at[jnp.arange(K1P), p1].add(
        jnp.where(jnp.arange(K1P) < K1, 1.0, 0.0).astype(bf))

    S1m, S1t, cs1 = _restricted_product(Rm, Ct, *RP1)
    deg1 = cs1[0] + 2.0
    dis1 = jax.lax.rsqrt(deg1)
    wvec1 = jnp.full((K1P,), 2.0)

    zs1, e1 = _z_producer(xp1, Wd1, bd1, dis1, wvec1)
    x2 = _aggregate(S1t, zs1, dis1, e1, relu=True)

    s2 = jnp.tanh((x2 @ attn1) / jnp.linalg.norm(attn1))
    s2 = jnp.where(jnp.arange(K1P) < K1, s2, -2.0)
    p2, v2 = _topk(s2, K2, K2P, K1P - 1)
    xp2 = x2[p2] * v2[:, None]

    Rp2 = _gather_diag(S1m, p2, K2, K2P)
    Ct2 = _gather_diag(S1t, p2, K2, K2P)
    S2m, S2t, cs2 = _restricted_product(Rp2, Ct2, *RP2)
    deg2 = cs2[0] + 2.0
    dis2 = jax.lax.rsqrt(deg2)
    wvec2 = jnp.full((K2P,), 2.0)

    zs2, e2 = _z_producer(xp2, Wd2, bd2, dis2, wvec2)
    x3 = _aggregate(S2t, zs2, dis2, e2, relu=True)

    s3 = jnp.tanh((x3 @ attn2) / jnp.linalg.norm(attn2))
    s3 = jnp.where(jnp.arange(K2P) < K2, s3, -2.0)
    p3, v3 = _topk(s3, K3, K3P, K2P - 1)
    xp3 = x3[p3] * v3[:, None]

    Rp3 = _gather_diag(S2m, p3, K3, K3P)
    Ct3 = _gather_diag(S2t, p3, K3, K3P)
    S3m, S3t, cs3 = _restricted_product(Rp3, Ct3, *RP3)
    deg3 = cs3[0] + 2.0
    dis3 = jax.lax.rsqrt(deg3)
    wvec3 = jnp.full((K3P,), 2.0)

    zs3, e3 = _z_producer(xp3, Wd3, bd3, dis3, wvec3)
    x4 = _aggregate(S3t, zs3, dis3, e3, relu=True)

    up3 = jnp.zeros((K2P, F), jnp.float32).at[p3].set(x4)
    zs5, e5 = _z_producer(jnp.concatenate([x3, up3], axis=1), Wu0, bu0,
                          dis2, wvec2)
    x5 = _aggregate(S2t, zs5, dis2, e5, relu=True)

    up2 = jnp.zeros((K1P, F), jnp.float32).at[p2].set(x5)
    zs6, e6 = _z_producer(jnp.concatenate([x2, up2], axis=1), Wu1, bu1,
                          dis1, wvec1)
    x6 = _aggregate(S1t, zs6, dis1, e6, relu=True)

    up1 = jnp.zeros((NP, F), jnp.float32).at[p1].set(x6)
    zs7, e7 = _z_producer(jnp.concatenate([x1, up1], axis=1), Wu2, bu2,
                          dinv0, wvec0)
    t7 = jnp.zeros((NP, F), jnp.float32).at[dst].add(zs7[src])
    x7 = dinv0[:, None] * t7 + e7

    return (x7[:N], edge_index)


# speedup vs baseline: 1.4853x; 1.4853x over previous
import functools
import math

import jax
import jax.numpy as jnp
from jax.experimental import pallas as pl
from jax.experimental.pallas import tpu as pltpu

N = 10000
NP = 10240
K1, K1P = 5000, 5120
K2, K2P = 2500, 2560
K3, K3P = 1250, 1280
F = 128
RP1 = (512, 512, 1024)
RP2 = (512, 512, 1024)
RP3 = (256, 256, 512)
ZBM = 512
ABM, ABK = 512, 256


def _rp_body(a_ref, b_ref, m_ref, mt_ref, deg_ref, acc_ref, *, nk, bm, bn):
    j = pl.program_id(0)
    i = pl.program_id(1)
    k = pl.program_id(2)

    @pl.when(k == 0)
    def _zero():
        acc_ref[...] = jnp.zeros_like(acc_ref)

    acc_ref[...] += jax.lax.dot_general(
        a_ref[...], b_ref[...], (((1,), (1,)), ((), ())),
        preferred_element_type=jnp.float32)

    @pl.when(k == nk - 1)
    def _fin():
        acc = acc_ref[...]
        rows = i * bm + jax.lax.broadcasted_iota(jnp.int32, (bm, bn), 0)
        cols = j * bn + jax.lax.broadcasted_iota(jnp.int32, (bm, bn), 1)
        acc = jnp.where(rows == cols, 0.0, acc)
        m_ref[...] = acc.astype(m_ref.dtype)
        mt_ref[...] = acc.T.astype(mt_ref.dtype)

        @pl.when(i == 0)
        def _zd():
            deg_ref[...] = jnp.zeros_like(deg_ref)

        deg_ref[...] += jnp.sum(acc, axis=0, keepdims=True)


def _restricted_product(a, bt, bm, bn, bk):
    M, K = a.shape
    Nc = bt.shape[0]
    assert M % bm == 0 and Nc % bn == 0 and K % bk == 0, (M, Nc, K, bm, bn, bk)
    nk = K // bk
    grid = (Nc // bn, M // bm, nk)
    return pl.pallas_call(
        functools.partial(_rp_body, nk=nk, bm=bm, bn=bn),
        grid=grid,
        in_specs=[
            pl.BlockSpec((bm, bk), lambda j, i, k: (i, k)),
            pl.BlockSpec((bn, bk), lambda j, i, k: (j, k)),
        ],
        out_specs=[
            pl.BlockSpec((bm, bn), lambda j, i, k: (i, j)),
            pl.BlockSpec((bn, bm), lambda j, i, k: (j, i)),
            pl.BlockSpec((1, bn), lambda j, i, k: (0, j)),
        ],
        out_shape=[
            jax.ShapeDtypeStruct((M, Nc), jnp.float32),
            jax.ShapeDtypeStruct((Nc, M), jnp.float32),
            jax.ShapeDtypeStruct((1, Nc), jnp.float32),
        ],
        scratch_shapes=[pltpu.VMEM((bm, bn), jnp.float32)],
    )(a, bt)


def _z_body(feat_ref, w_ref, b_ref, dis_ref, wv_ref, zs_ref, e_ref):
    z = jnp.dot(feat_ref[...], w_ref[...], preferred_element_type=jnp.float32)
    dis = dis_ref[...]
    zs = dis * z
    zs_ref[...] = zs
    e_ref[...] = wv_ref[...] * dis * zs + b_ref[...]


def _z_producer(feat, w, b, dis, wvec):
    M, Fin = feat.shape
    bm = ZBM if M % ZBM == 0 else 256
    assert M % bm == 0, (M, bm)
    return pl.pallas_call(
        _z_body,
        grid=(M // bm,),
        in_specs=[
            pl.BlockSpec((bm, Fin), lambda i: (i, 0)),
            pl.BlockSpec((Fin, F), lambda i: (0, 0)),
            pl.BlockSpec((1, F), lambda i: (0, 0)),
            pl.BlockSpec((bm, 1), lambda i: (i, 0)),
            pl.BlockSpec((bm, 1), lambda i: (i, 0)),
        ],
        out_specs=[
            pl.BlockSpec((bm, F), lambda i: (i, 0)),
            pl.BlockSpec((bm, F), lambda i: (i, 0)),
        ],
        out_shape=[
            jax.ShapeDtypeStruct((M, F), jnp.float32),
            jax.ShapeDtypeStruct((M, F), jnp.float32),
        ],
    )(feat, w, b.reshape(1, F), dis.reshape(M, 1), wvec.reshape(M, 1))


def _agg_body(mt_ref, zs_ref, dis_ref, e_ref, out_ref, acc_ref, *, nk, relu):
    k = pl.program_id(1)

    @pl.when(k == 0)
    def _zero():
        acc_ref[...] = jnp.zeros_like(acc_ref)

    acc_ref[...] += jnp.dot(mt_ref[...], zs_ref[...],
                            preferred_element_type=jnp.float32)

    @pl.when(k == nk - 1)
    def _fin():
        o = dis_ref[...] * acc_ref[...] + e_ref[...]
        if relu:
            o = jnp.maximum(o, 0.0)
        out_ref[...] = o


def _aggregate(mt, zs, dis, e, relu):
    M, K = mt.shape
    bm = ABM if M % ABM == 0 else 256
    bk = ABK
    assert M % bm == 0 and K % bk == 0, (M, K, bm, bk)
    nk = K // bk
    return pl.pallas_call(
        functools.partial(_agg_body, nk=nk, relu=relu),
        grid=(M // bm, nk),
        in_specs=[
            pl.BlockSpec((bm, bk), lambda i, k: (i, k)),
            pl.BlockSpec((bk, F), lambda i, k: (k, 0)),
            pl.BlockSpec((bm, 1), lambda i, k: (i, 0)),
            pl.BlockSpec((bm, F), lambda i, k: (i, 0)),
        ],
        out_specs=pl.BlockSpec((bm, F), lambda i, k: (i, 0)),
        out_shape=jax.ShapeDtypeStruct((M, F), jnp.float32),
        scratch_shapes=[pltpu.VMEM((bm, F), jnp.float32)],
    )(mt, zs, dis.reshape(M, 1), e)


def _topk(score, k, kp, dummy_idx):
    thr = jax.lax.top_k(score, k)[0][-1]
    mask = score >= thr
    perm = jnp.nonzero(mask, size=kp, fill_value=dummy_idx)[0].astype(jnp.int32)
    valid = jnp.arange(kp) < k
    vals = jnp.where(valid, score[perm], 0.0)
    return perm, vals


def _gather_diag(s_mat, perm, k, kp):
    g = s_mat[perm]
    ind = jnp.where(jnp.arange(kp) < k, 1.0, 0.0).astype(s_mat.dtype)
    return g.at[jnp.arange(kp), perm].add(ind)


def kernel(x, edge_index, Wd0, bd0, Wd1, bd1, Wd2, bd2, Wd3, bd3,
           attn0, attn1, attn2, Wu0, bu0, Wu1, bu1, Wu2, bu2):
    src, dst = edge_index[0], edge_index[1]

    xpad = jnp.zeros((NP, F), jnp.float32).at[:N].set(x)

    ones_e = jnp.ones_like(src, dtype=jnp.float32)
    indeg = jnp.zeros((NP,), jnp.float32).at[dst].add(ones_e)
    selfcnt = jnp.zeros((NP,), jnp.float32).at[dst].add(
        jnp.where(src == dst, 1.0, 0.0))
    deg0 = indeg + jnp.where(selfcnt > 0, 0.0, 2.0)
    dinv0 = jax.lax.rsqrt(deg0)
    wvec0 = jnp.where(selfcnt > 0, 0.0, 2.0)

    zs0, e0 = _z_producer(xpad, Wd0, bd0, dinv0, wvec0)
    t0 = jnp.zeros((NP, F), jnp.float32).at[dst].add(zs0[src])
    x1 = jax.nn.relu(dinv0[:, None] * t0 + e0)

    s1 = jnp.tanh((x1 @ attn0) / jnp.linalg.norm(attn0))
    s1 = jnp.where(jnp.arange(NP) < N, s1, -2.0)
    p1, v1 = _topk(s1, K1, K1P, NP - 1)
    xp1 = x1[p1] * v1[:, None]

    inv1 = jnp.full((NP,), -1, jnp.int32).at[p1].set(
        jnp.arange(K1P, dtype=jnp.int32))
    offd = src != dst
    msrc = offd & (inv1[src] >= 0) & (inv1[src] < K1)
    Rm = jnp.zeros((K1P, NP), jnp.float32).at[
        jnp.where(msrc, inv1[src], 0), dst].add(jnp.where(msrc, 1.0, 0.0))
    Rm = Rm.at[jnp.arange(K1P), p1].add(jnp.where(jnp.arange(K1P) < K1, 1.0, 0.0))
    mdst = offd & (inv1[dst] >= 0) & (inv1[dst] < K1)
    Ct = jnp.zeros((K1P, NP), jnp.float32).at[
        jnp.where(mdst, inv1[dst], 0), src].add(jnp.where(mdst, 1.0, 0.0))
    Ct = Ct.at[jnp.arange(K1P), p1].add(jnp.where(jnp.arange(K1P) < K1, 1.0, 0.0))

    S1m, S1t, cs1 = _restricted_product(Rm.astype(jnp.bfloat16), Ct.astype(jnp.bfloat16), *RP1)
    deg1 = cs1[0] + 2.0
    dis1 = jax.lax.rsqrt(deg1)
    wvec1 = jnp.full((K1P,), 2.0)

    zs1, e1 = _z_producer(xp1, Wd1, bd1, dis1, wvec1)
    x2 = _aggregate(S1t, zs1, dis1, e1, relu=True)

    s2 = jnp.tanh((x2 @ attn1) / jnp.linalg.norm(attn1))
    s2 = jnp.where(jnp.arange(K1P) < K1, s2, -2.0)
    p2, v2 = _topk(s2, K2, K2P, K1P - 1)
    xp2 = x2[p2] * v2[:, None]

    Rp2 = _gather_diag(S1m, p2, K2, K2P)
    Ct2 = _gather_diag(S1t, p2, K2, K2P)
    S2m, S2t, cs2 = _restricted_product(Rp2.astype(jnp.bfloat16), Ct2.astype(jnp.bfloat16), *RP2)
    deg2 = cs2[0] + 2.0
    dis2 = jax.lax.rsqrt(deg2)
    wvec2 = jnp.full((K2P,), 2.0)

    zs2, e2 = _z_producer(xp2, Wd2, bd2, dis2, wvec2)
    x3 = _aggregate(S2t, zs2, dis2, e2, relu=True)

    s3 = jnp.tanh((x3 @ attn2) / jnp.linalg.norm(attn2))
    s3 = jnp.where(jnp.arange(K2P) < K2, s3, -2.0)
    p3, v3 = _topk(s3, K3, K3P, K2P - 1)
    xp3 = x3[p3] * v3[:, None]

    Rp3 = _gather_diag(S2m, p3, K3, K3P)
    Ct3 = _gather_diag(S2t, p3, K3, K3P)
    S3m, S3t, cs3 = _restricted_product(Rp3.astype(jnp.bfloat16), Ct3.astype(jnp.bfloat16), *RP3)
    deg3 = cs3[0] + 2.0
    dis3 = jax.lax.rsqrt(deg3)
    wvec3 = jnp.full((K3P,), 2.0)

    zs3, e3 = _z_producer(xp3, Wd3, bd3, dis3, wvec3)
    x4 = _aggregate(S3t, zs3, dis3, e3, relu=True)

    up3 = jnp.zeros((K2P, F), jnp.float32).at[p3].set(x4)
    zs5, e5 = _z_producer(jnp.concatenate([x3, up3], axis=1), Wu0, bu0,
                          dis2, wvec2)
    x5 = _aggregate(S2t, zs5, dis2, e5, relu=True)

    up2 = jnp.zeros((K1P, F), jnp.float32).at[p2].set(x5)
    zs6, e6 = _z_producer(jnp.concatenate([x2, up2], axis=1), Wu1, bu1,
                          dis1, wvec1)
    x6 = _aggregate(S1t, zs6, dis1, e6, relu=True)

    up1 = jnp.zeros((NP, F), jnp.float32).at[p1].set(x6)
    zs7, e7 = _z_producer(jnp.concatenate([x1, up1], axis=1), Wu2, bu2,
                          dinv0, wvec0)
    t7 = jnp.zeros((NP, F), jnp.float32).at[dst].add(zs7[src])
    x7 = dinv0[:, None] * t7 + e7

    return (x7[:N], edge_index)


# speedup vs baseline: 1.4855x; 1.0002x over previous
import functools
import math

import jax
import jax.numpy as jnp
from jax.experimental import pallas as pl
from jax.experimental.pallas import tpu as pltpu

N = 10000
NP = 10240
K1, K1P = 5000, 5120
K2, K2P = 2500, 2560
K3, K3P = 1250, 1280
F = 128
RP1 = (512, 512, 1024)
RP2 = (512, 512, 1024)
RP3 = (256, 256, 512)
ZBM = 512
ABM, ABK = 512, 256


def _rp_body(a_ref, b_ref, m_ref, mt_ref, deg_ref, acc_ref, *, nk, bm, bn):
    j = pl.program_id(0)
    i = pl.program_id(1)
    k = pl.program_id(2)

    @pl.when(k == 0)
    def _zero():
        acc_ref[...] = jnp.zeros_like(acc_ref)

    acc_ref[...] += jax.lax.dot_general(
        a_ref[...], b_ref[...], (((1,), (1,)), ((), ())),
        preferred_element_type=jnp.float32)

    @pl.when(k == nk - 1)
    def _fin():
        acc = acc_ref[...]
        rows = i * bm + jax.lax.broadcasted_iota(jnp.int32, (bm, bn), 0)
        cols = j * bn + jax.lax.broadcasted_iota(jnp.int32, (bm, bn), 1)
        acc = jnp.where(rows == cols, 0.0, acc)
        m_ref[...] = acc.astype(m_ref.dtype)
        mt_ref[...] = acc.T.astype(mt_ref.dtype)

        @pl.when(i == 0)
        def _zd():
            deg_ref[...] = jnp.zeros_like(deg_ref)

        deg_ref[...] += jnp.sum(acc, axis=0, keepdims=True)


def _restricted_product(a, bt, bm, bn, bk):
    M, K = a.shape
    Nc = bt.shape[0]
    assert M % bm == 0 and Nc % bn == 0 and K % bk == 0, (M, Nc, K, bm, bn, bk)
    nk = K // bk
    grid = (Nc // bn, M // bm, nk)
    return pl.pallas_call(
        functools.partial(_rp_body, nk=nk, bm=bm, bn=bn),
        grid=grid,
        in_specs=[
            pl.BlockSpec((bm, bk), lambda j, i, k: (i, k)),
            pl.BlockSpec((bn, bk), lambda j, i, k: (j, k)),
        ],
        out_specs=[
            pl.BlockSpec((bm, bn), lambda j, i, k: (i, j)),
            pl.BlockSpec((bn, bm), lambda j, i, k: (j, i)),
            pl.BlockSpec((1, bn), lambda j, i, k: (0, j)),
        ],
        out_shape=[
            jax.ShapeDtypeStruct((M, Nc), jnp.float32),
            jax.ShapeDtypeStruct((Nc, M), jnp.float32),
            jax.ShapeDtypeStruct((1, Nc), jnp.float32),
        ],
        scratch_shapes=[pltpu.VMEM((bm, bn), jnp.float32)],
    )(a, bt)


def _z_body(feat_ref, w_ref, b_ref, dis_ref, wv_ref, zs_ref, e_ref):
    z = jnp.dot(feat_ref[...], w_ref[...], preferred_element_type=jnp.float32)
    dis = dis_ref[...]
    zs = dis * z
    zs_ref[...] = zs
    e_ref[...] = wv_ref[...] * dis * zs + b_ref[...]


def _z_producer(feat, w, b, dis, wvec):
    M, Fin = feat.shape
    bm = ZBM if M % ZBM == 0 else 256
    assert M % bm == 0, (M, bm)
    return pl.pallas_call(
        _z_body,
        grid=(M // bm,),
        in_specs=[
            pl.BlockSpec((bm, Fin), lambda i: (i, 0)),
            pl.BlockSpec((Fin, F), lambda i: (0, 0)),
            pl.BlockSpec((1, F), lambda i: (0, 0)),
            pl.BlockSpec((bm, 1), lambda i: (i, 0)),
            pl.BlockSpec((bm, 1), lambda i: (i, 0)),
        ],
        out_specs=[
            pl.BlockSpec((bm, F), lambda i: (i, 0)),
            pl.BlockSpec((bm, F), lambda i: (i, 0)),
        ],
        out_shape=[
            jax.ShapeDtypeStruct((M, F), jnp.float32),
            jax.ShapeDtypeStruct((M, F), jnp.float32),
        ],
    )(feat, w, b.reshape(1, F), dis.reshape(M, 1), wvec.reshape(M, 1))


def _agg_body(mt_ref, zs_ref, dis_ref, e_ref, out_ref, acc_ref, *, nk, relu):
    k = pl.program_id(1)

    @pl.when(k == 0)
    def _zero():
        acc_ref[...] = jnp.zeros_like(acc_ref)

    acc_ref[...] += jnp.dot(mt_ref[...], zs_ref[...],
                            preferred_element_type=jnp.float32)

    @pl.when(k == nk - 1)
    def _fin():
        o = dis_ref[...] * acc_ref[...] + e_ref[...]
        if relu:
            o = jnp.maximum(o, 0.0)
        out_ref[...] = o


def _aggregate(mt, zs, dis, e, relu):
    M, K = mt.shape
    bm = ABM if M % ABM == 0 else 256
    bk = ABK
    assert M % bm == 0 and K % bk == 0, (M, K, bm, bk)
    nk = K // bk
    return pl.pallas_call(
        functools.partial(_agg_body, nk=nk, relu=relu),
        grid=(M // bm, nk),
        in_specs=[
            pl.BlockSpec((bm, bk), lambda i, k: (i, k)),
            pl.BlockSpec((bk, F), lambda i, k: (k, 0)),
            pl.BlockSpec((bm, 1), lambda i, k: (i, 0)),
            pl.BlockSpec((bm, F), lambda i, k: (i, 0)),
        ],
        out_specs=pl.BlockSpec((bm, F), lambda i, k: (i, 0)),
        out_shape=jax.ShapeDtypeStruct((M, F), jnp.float32),
        scratch_shapes=[pltpu.VMEM((bm, F), jnp.float32)],
    )(mt, zs, dis.reshape(M, 1), e)


def _thr_body(s_ref, thr_ref, *, k):
    s = s_ref[...]
    b = jax.lax.bitcast_convert_type(jnp.abs(s), jnp.int32)
    key = jnp.where(s < 0.0, -b, b)

    def step(_, lohi):
        lo, hi = lohi
        mid = lo + (hi - lo) // 2
        cnt = jnp.sum((key >= mid).astype(jnp.int32))
        big = cnt >= k
        return (jnp.where(big, mid, lo), jnp.where(big, hi, mid))

    lo0 = jnp.int32(-(2 ** 30))
    hi0 = jnp.int32(2 ** 30 - 1)
    lo, hi = jax.lax.fori_loop(0, 32, step, (lo0, hi0))
    athr = jax.lax.bitcast_convert_type(jnp.abs(lo), jnp.float32)
    thr = jnp.where(lo < 0, -athr, athr)
    thr_ref[...] = jnp.full_like(thr_ref, thr)


def _topk_threshold(score, k):
    n = score.shape[0]
    out = pl.pallas_call(
        functools.partial(_thr_body, k=k),
        in_specs=[pl.BlockSpec((1, n), lambda: (0, 0))],
        out_specs=pl.BlockSpec((1, 128), lambda: (0, 0)),
        out_shape=jax.ShapeDtypeStruct((1, 128), jnp.float32),
    )(score.reshape(1, n))
    return out[0, 0]


def _topk(score, k, kp, dummy_idx):
    thr = _topk_threshold(score, k)
    mask = score >= thr
    perm = jnp.nonzero(mask, size=kp, fill_value=dummy_idx)[0].astype(jnp.int32)
    valid = jnp.arange(kp) < k
    vals = jnp.where(valid, score[perm], 0.0)
    return perm, vals


def _gather_diag(s_mat, perm, k, kp):
    g = s_mat[perm]
    ind = jnp.where(jnp.arange(kp) < k, 1.0, 0.0).astype(s_mat.dtype)
    return g.at[jnp.arange(kp), perm].add(ind)


def kernel(x, edge_index, Wd0, bd0, Wd1, bd1, Wd2, bd2, Wd3, bd3,
           attn0, attn1, attn2, Wu0, bu0, Wu1, bu1, Wu2, bu2):
    src, dst = edge_index[0], edge_index[1]

    xpad = jnp.zeros((NP, F), jnp.float32).at[:N].set(x)

    ones_e = jnp.ones_like(src, dtype=jnp.float32)
    indeg = jnp.zeros((NP,), jnp.float32).at[dst].add(ones_e)
    selfcnt = jnp.zeros((NP,), jnp.float32).at[dst].add(
        jnp.where(src == dst, 1.0, 0.0))
    deg0 = indeg + jnp.where(selfcnt > 0, 0.0, 2.0)
    dinv0 = jax.lax.rsqrt(deg0)
    wvec0 = jnp.where(selfcnt > 0, 0.0, 2.0)

    zs0, e0 = _z_producer(xpad, Wd0, bd0, dinv0, wvec0)
    t0 = jnp.zeros((NP, F), jnp.float32).at[dst].add(zs0[src])
    x1 = jax.nn.relu(dinv0[:, None] * t0 + e0)

    s1 = jnp.tanh((x1 @ attn0) / jnp.linalg.norm(attn0))
    s1 = jnp.where(jnp.arange(NP) < N, s1, -2.0)
    p1, v1 = _topk(s1, K1, K1P, NP - 1)
    xp1 = x1[p1] * v1[:, None]

    inv1 = jnp.full((NP,), -1, jnp.int32).at[p1].set(
        jnp.arange(K1P, dtype=jnp.int32))
    offd = src != dst
    msrc = offd & (inv1[src] >= 0) & (inv1[src] < K1)
    Rm = jnp.zeros((K1P, NP), jnp.float32).at[
        jnp.where(msrc, inv1[src], 0), dst].add(jnp.where(msrc, 1.0, 0.0))
    Rm = Rm.at[jnp.arange(K1P), p1].add(jnp.where(jnp.arange(K1P) < K1, 1.0, 0.0))
    mdst = offd & (inv1[dst] >= 0) & (inv1[dst] < K1)
    Ct = jnp.zeros((K1P, NP), jnp.float32).at[
        jnp.where(mdst, inv1[dst], 0), src].add(jnp.where(mdst, 1.0, 0.0))
    Ct = Ct.at[jnp.arange(K1P), p1].add(jnp.where(jnp.arange(K1P) < K1, 1.0, 0.0))

    S1m, S1t, cs1 = _restricted_product(Rm.astype(jnp.bfloat16), Ct.astype(jnp.bfloat16), *RP1)
    deg1 = cs1[0] + 2.0
    dis1 = jax.lax.rsqrt(deg1)
    wvec1 = jnp.full((K1P,), 2.0)

    zs1, e1 = _z_producer(xp1, Wd1, bd1, dis1, wvec1)
    x2 = _aggregate(S1t, zs1, dis1, e1, relu=True)

    s2 = jnp.tanh((x2 @ attn1) / jnp.linalg.norm(attn1))
    s2 = jnp.where(jnp.arange(K1P) < K1, s2, -2.0)
    p2, v2 = _topk(s2, K2, K2P, K1P - 1)
    xp2 = x2[p2] * v2[:, None]

    Rp2 = _gather_diag(S1m, p2, K2, K2P)
    Ct2 = _gather_diag(S1t, p2, K2, K2P)
    S2m, S2t, cs2 = _restricted_product(Rp2.astype(jnp.bfloat16), Ct2.astype(jnp.bfloat16), *RP2)
    deg2 = cs2[0] + 2.0
    dis2 = jax.lax.rsqrt(deg2)
    wvec2 = jnp.full((K2P,), 2.0)

    zs2, e2 = _z_producer(xp2, Wd2, bd2, dis2, wvec2)
    x3 = _aggregate(S2t, zs2, dis2, e2, relu=True)

    s3 = jnp.tanh((x3 @ attn2) / jnp.linalg.norm(attn2))
    s3 = jnp.where(jnp.arange(K2P) < K2, s3, -2.0)
    p3, v3 = _topk(s3, K3, K3P, K2P - 1)
    xp3 = x3[p3] * v3[:, None]

    Rp3 = _gather_diag(S2m, p3, K3, K3P)
    Ct3 = _gather_diag(S2t, p3, K3, K3P)
    S3m, S3t, cs3 = _restricted_product(Rp3.astype(jnp.bfloat16), Ct3.astype(jnp.bfloat16), *RP3)
    deg3 = cs3[0] + 2.0
    dis3 = jax.lax.rsqrt(deg3)
    wvec3 = jnp.full((K3P,), 2.0)

    zs3, e3 = _z_producer(xp3, Wd3, bd3, dis3, wvec3)
    x4 = _aggregate(S3t, zs3, dis3, e3, relu=True)

    up3 = jnp.zeros((K2P, F), jnp.float32).at[p3].set(x4)
    zs5, e5 = _z_producer(jnp.concatenate([x3, up3], axis=1), Wu0, bu0,
                          dis2, wvec2)
    x5 = _aggregate(S2t, zs5, dis2, e5, relu=True)

    up2 = jnp.zeros((K1P, F), jnp.float32).at[p2].set(x5)
    zs6, e6 = _z_producer(jnp.concatenate([x2, up2], axis=1), Wu1, bu1,
                          dis1, wvec1)
    x6 = _aggregate(S1t, zs6, dis1, e6, relu=True)

    up1 = jnp.zeros((NP, F), jnp.float32).at[p1].set(x6)
    zs7, e7 = _z_producer(jnp.concatenate([x1, up1], axis=1), Wu2, bu2,
                          dinv0, wvec0)
    t7 = jnp.zeros((NP, F), jnp.float32).at[dst].add(zs7[src])
    x7 = dinv0[:, None] * t7 + e7

    return (x7[:N], edge_index)
